# baseline (speedup 1.0000x reference)
import numpy as np

import jax
import jax.numpy as jnp
from jax.experimental import pallas as pl
from jax.experimental.pallas import tpu as pltpu

_NEG = 0.01
_BT = 2048


def _band(n_out, n_in, n_tap):
    e = np.zeros((n_out, n_in, n_tap), np.float32)
    for b in range(n_in):
        for d in range(n_tap):
            e[b + d, b, d] = 1.0
    return e


_E21 = _band(21, 17, 5)
_E4 = _band(4, 2, 3)
_E17 = _band(17, 15, 3)


def _conv1_operator(conv1_w, conv1_b):
    w1 = conv1_w.reshape(6, 5, 5).astype(jnp.float32)
    tmp = jnp.einsum("rjd,oad->rjoa", _E21, w1)
    m = jnp.einsum("xia,yjoa->xyijo", _E21, tmp)
    m = m.reshape(441, 17, 102).astype(jnp.bfloat16)
    m = jnp.pad(m, ((0, 0), (0, 0), (0, 26)))
    bias = jnp.broadcast_to(conv1_b.astype(jnp.bfloat16)[None, None, :],
                            (1, 289, 6)).reshape(1, 17, 17, 6)
    bias = jnp.pad(bias.reshape(1, 17, 102), ((0, 0), (0, 0), (0, 26)))
    return jnp.concatenate([m, bias], axis=0).reshape(442, 2176)


def _conv2_operator(conv2_w):
    w2 = conv2_w.astype(jnp.float32)
    tmp = jnp.einsum("pjd,ocad->pjoca", _E17, w2)
    m = jnp.einsum("rta,pjoca->rpctjo", _E4, tmp)
    m = m.reshape(4, 102, 2, 5, 3, 16).astype(jnp.bfloat16)
    m = jnp.pad(m, ((0, 0), (0, 26), (0, 0), (0, 0), (0, 0), (0, 0)))
    m = m.transpose(4, 0, 1, 2, 3, 5)
    m = m.reshape(3, 4, 128, 2, 80)
    m = jnp.pad(m, ((0, 0), (0, 0), (0, 0), (0, 0), (0, 48)))
    return m.transpose(1, 2, 0, 3, 4).reshape(512, 768)


def _leaky(x):
    return jnp.maximum(x, x * _NEG)


def _body(x_ref, m1_ref, m2s_ref, b2_ref, f1_ref, f2_ref, f3_ref, o_ref):
    x = jnp.pad(x_ref[...], ((0, 0), (0, 1)), constant_values=1)
    h1 = jnp.dot(x, m1_ref[...], preferred_element_type=jnp.float32)
    h1 = _leaky(h1.astype(jnp.bfloat16))

    vp = [None] * 5

    def fold(i, blk):
        g = i // 3
        vp[g] = blk if vp[g] is None else jnp.maximum(vp[g], blk)

    m2s = m2s_ref[...]
    for p in range(7):
        acc = jnp.dot(h1[:, 256 * p:256 * p + 512], m2s,
                      preferred_element_type=jnp.float32)
        am = jnp.maximum(jnp.maximum(acc[:, :256], acc[:, 256:512]),
                         acc[:, 512:768])
        fold(2 * p, am[:, :128])
        fold(2 * p + 1, am[:, 128:])
    m2l = jnp.concatenate([m2s[:384, 256 * s:256 * s + 128] for s in range(3)],
                          axis=1)
    acc = jnp.dot(h1[:, 1792:2176], m2l,
                  preferred_element_type=jnp.float32)
    fold(14, jnp.maximum(jnp.maximum(acc[:, :128], acc[:, 128:256]),
                         acc[:, 256:384]))

    f = jnp.concatenate(vp, axis=1) + b2_ref[...]
    f = _leaky(f).astype(jnp.bfloat16)

    h = jnp.dot(f, f1_ref[...], preferred_element_type=jnp.float32)
    h = _leaky(h).astype(jnp.bfloat16)
    h = jnp.dot(h, f2_ref[...], preferred_element_type=jnp.float32)
    h = _leaky(h).astype(jnp.bfloat16)
    o_ref[...] = jnp.dot(h, f3_ref[...], preferred_element_type=jnp.float32)


def kernel(a, c, conv1_w, conv1_b, conv2_w, conv2_b, fc1_w, fc2_w, fc3_w):
    B = a.shape[0]
    od = fc3_w.shape[1]
    bt = _BT if B >= _BT else B
    bp = ((B + bt - 1) // bt) * bt

    x = a.reshape(B, 441).astype(jnp.bfloat16)
    if bp != B:
        x = jnp.pad(x, ((0, bp - B), (0, 0)))

    m1 = _conv1_operator(conv1_w, conv1_b)
    m2s = _conv2_operator(conv2_w)
    b2row = jnp.broadcast_to(conv2_b.astype(jnp.float32)[None, None, None, :],
                             (1, 5, 5, 16)).reshape(1, 5, 80)
    b2row = jnp.pad(b2row, ((0, 0), (0, 0), (0, 48))).reshape(1, 640)
    f1 = fc1_w.astype(jnp.bfloat16).reshape(16, 5, 5, 128).transpose(1, 2, 0, 3) \
        .reshape(5, 80, 128)
    f1 = jnp.pad(f1, ((0, 0), (0, 48), (0, 0))).reshape(640, 128)
    f2 = fc2_w.astype(jnp.bfloat16)
    f3 = fc3_w.astype(jnp.bfloat16)

    out = pl.pallas_call(
        _body,
        out_shape=jax.ShapeDtypeStruct((bp, od), jnp.float32),
        grid=(bp // bt,),
        in_specs=[
            pl.BlockSpec((bt, 441), lambda i: (i, 0)),
            pl.BlockSpec((442, 2176), lambda i: (0, 0)),
            pl.BlockSpec((512, 768), lambda i: (0, 0)),
            pl.BlockSpec((1, 640), lambda i: (0, 0)),
            pl.BlockSpec((640, 128), lambda i: (0, 0)),
            pl.BlockSpec((128, 64), lambda i: (0, 0)),
            pl.BlockSpec((64, od), lambda i: (0, 0)),
        ],
        out_specs=pl.BlockSpec((bt, od), lambda i: (i, 0)),
        compiler_params=pltpu.CompilerParams(
            dimension_semantics=("parallel",)),
    )(x, m1, m2s, b2row, f1, f2, f3)
    return out[:B]


# speedup vs baseline: 1.0258x; 1.0258x over previous
import numpy as np

import jax
import jax.numpy as jnp
from jax.experimental import pallas as pl
from jax.experimental.pallas import tpu as pltpu

_NEG = 0.01
_BT = 2048


def _band(n_out, n_in, n_tap):
    e = np.zeros((n_out, n_in, n_tap), np.float32)
    for b in range(n_in):
        for d in range(n_tap):
            e[b + d, b, d] = 1.0
    return e


_E21 = _band(21, 17, 5)
_E4 = _band(4, 2, 3)
_E17 = _band(17, 15, 3)


def _conv1_operator(conv1_w, conv1_b):
    w1 = conv1_w.reshape(6, 5, 5).astype(jnp.float32)
    tmp = jnp.einsum("rjd,oad->rjoa", _E21, w1)
    m = jnp.einsum("xia,yjoa->xyijo", _E21, tmp)
    m = m.reshape(441, 17, 102).astype(jnp.bfloat16)
    m = jnp.pad(m, ((0, 0), (0, 0), (0, 26)))
    bias = jnp.broadcast_to(conv1_b.astype(jnp.bfloat16)[None, None, :],
                            (1, 289, 6)).reshape(1, 17, 17, 6)
    bias = jnp.pad(bias.reshape(1, 17, 102), ((0, 0), (0, 0), (0, 26)))
    return jnp.concatenate([m, bias], axis=0).reshape(442, 2176)


def _conv2_operator(conv2_w):
    w2 = conv2_w.astype(jnp.float32)
    tmp = jnp.einsum("pjd,ocad->pjoca", _E17, w2)
    m = jnp.einsum("rta,pjoca->rpctjo", _E4, tmp)
    m = m.reshape(4, 102, 2, 5, 3, 16).astype(jnp.bfloat16)
    m = jnp.pad(m, ((0, 0), (0, 26), (0, 0), (0, 0), (0, 0), (0, 0)))
    m = m.transpose(4, 0, 1, 2, 3, 5)
    m = m.reshape(3, 4, 128, 2, 80)
    m = jnp.pad(m, ((0, 0), (0, 0), (0, 0), (0, 0), (0, 48)))
    return m.transpose(1, 2, 0, 3, 4).reshape(512, 768)


def _leaky(x):
    return jnp.maximum(x, x * _NEG)


def _body(x_ref, m1_ref, m2s_ref, b2_ref, f1_ref, f2_ref, f3_ref, o_ref):
    x = jnp.pad(x_ref[...].astype(jnp.bfloat16), ((0, 0), (0, 1)),
                constant_values=1)
    h1 = jnp.dot(x, m1_ref[...], preferred_element_type=jnp.float32)
    h1 = _leaky(h1.astype(jnp.bfloat16))

    vp = [None] * 5

    def fold(i, blk):
        g = i // 3
        vp[g] = blk if vp[g] is None else jnp.maximum(vp[g], blk)

    m2s = m2s_ref[...]
    for p in range(7):
        acc = jnp.dot(h1[:, 256 * p:256 * p + 512], m2s,
                      preferred_element_type=jnp.float32)
        am = jnp.maximum(jnp.maximum(acc[:, :256], acc[:, 256:512]),
                         acc[:, 512:768])
        fold(2 * p, am[:, :128])
        fold(2 * p + 1, am[:, 128:])
    m2l = jnp.concatenate([m2s[:384, 256 * s:256 * s + 128] for s in range(3)],
                          axis=1)
    acc = jnp.dot(h1[:, 1792:2176], m2l,
                  preferred_element_type=jnp.float32)
    fold(14, jnp.maximum(jnp.maximum(acc[:, :128], acc[:, 128:256]),
                         acc[:, 256:384]))

    f = jnp.concatenate(vp, axis=1) + b2_ref[...]
    f = _leaky(f).astype(jnp.bfloat16)

    h = jnp.dot(f, f1_ref[...], preferred_element_type=jnp.float32)
    h = _leaky(h).astype(jnp.bfloat16)
    h = jnp.dot(h, f2_ref[...], preferred_element_type=jnp.float32)
    h = _leaky(h).astype(jnp.bfloat16)
    o_ref[...] = jnp.dot(h, f3_ref[...], preferred_element_type=jnp.float32)


def kernel(a, c, conv1_w, conv1_b, conv2_w, conv2_b, fc1_w, fc2_w, fc3_w):
    B = a.shape[0]
    od = fc3_w.shape[1]
    bt = _BT if B >= _BT else B
    bp = ((B + bt - 1) // bt) * bt

    x = a.reshape(B, 441)
    if bp != B:
        x = jnp.pad(x, ((0, bp - B), (0, 0)))

    m1 = _conv1_operator(conv1_w, conv1_b)
    m2s = _conv2_operator(conv2_w)
    b2row = jnp.broadcast_to(conv2_b.astype(jnp.float32)[None, None, None, :],
                             (1, 5, 5, 16)).reshape(1, 5, 80)
    b2row = jnp.pad(b2row, ((0, 0), (0, 0), (0, 48))).reshape(1, 640)
    f1 = fc1_w.astype(jnp.bfloat16).reshape(16, 5, 5, 128).transpose(1, 2, 0, 3) \
        .reshape(5, 80, 128)
    f1 = jnp.pad(f1, ((0, 0), (0, 48), (0, 0))).reshape(640, 128)
    f2 = fc2_w.astype(jnp.bfloat16)
    f3 = fc3_w.astype(jnp.bfloat16)

    out = pl.pallas_call(
        _body,
        out_shape=jax.ShapeDtypeStruct((bp, od), jnp.float32),
        grid=(bp // bt,),
        in_specs=[
            pl.BlockSpec((bt, 441), lambda i: (i, 0)),
            pl.BlockSpec((442, 2176), lambda i: (0, 0)),
            pl.BlockSpec((512, 768), lambda i: (0, 0)),
            pl.BlockSpec((1, 640), lambda i: (0, 0)),
            pl.BlockSpec((640, 128), lambda i: (0, 0)),
            pl.BlockSpec((128, 64), lambda i: (0, 0)),
            pl.BlockSpec((64, od), lambda i: (0, 0)),
        ],
        out_specs=pl.BlockSpec((bt, od), lambda i: (i, 0)),
        compiler_params=pltpu.CompilerParams(
            dimension_semantics=("parallel",)),
    )(x, m1, m2s, b2row, f1, f2, f3)
    return out[:B]
